# baseline (speedup 1.0000x reference)
import functools

import jax
import jax.numpy as jnp
from jax import lax
from jax.experimental import pallas as pl
from jax.experimental.pallas import tpu as pltpu
from jax.experimental.pallas import tpu_sc as plsc

N_CLS = 100
IMG_ELEMS = 3 * 224 * 224
B = 256
C = 12
DC = IMG_ELEMS // C
NW = 32
ROWS_PER_W = (B * C) // NW
K = 8
GROUPS = ROWS_PER_W // K
VECS = DC // 16


def _sc_body(table_hbm, sidx_hbm, idx_hbm, clstab_hbm, out_hbm, cls_hbm,
             sidx_v, buf, idx16_v, cls_v, sem):
    c = lax.axis_index("c")
    s = lax.axis_index("s")
    wid = s * 2 + c

    @pl.when(wid < 16)
    def _():
        pltpu.sync_copy(idx_hbm.at[pl.ds(wid * 16, 16)], idx16_v)
        pltpu.async_copy(clstab_hbm.at[idx16_v], cls_v, sem).wait()
        pltpu.sync_copy(cls_v, cls_hbm.at[pl.ds(wid * 16, 16)])

    base = wid * ROWS_PER_W
    pltpu.sync_copy(sidx_hbm.at[pl.ds(base, ROWS_PER_W)], sidx_v)
    for g in range(GROUPS):
        pltpu.async_copy(
            table_hbm.at[sidx_v.at[pl.ds(g * K, K)]], buf, sem
        ).wait()

        for r in range(K):
            def body(i, _):
                x = buf[r, pl.ds(i * 16, 16)]
                buf[r, pl.ds(i * 16, 16)] = (
                    jnp.minimum(jnp.maximum(x, -0.5), 0.5) + 0.5
                )
                return 0
            lax.fori_loop(0, VECS, body, 0)

        pltpu.sync_copy(buf, out_hbm.at[pl.ds(base + g * K, K)])


@jax.jit
def _run(table2, sidx, idx, clstab):
    mesh = plsc.VectorSubcoreMesh(
        core_axis_name="c", subcore_axis_name="s", num_cores=2, num_subcores=16
    )
    f = pl.kernel(
        _sc_body,
        out_type=(
            jax.ShapeDtypeStruct((B * C, DC), jnp.float32),
            jax.ShapeDtypeStruct((B,), jnp.int32),
        ),
        mesh=mesh,
        scratch_types=[
            pltpu.VMEM((ROWS_PER_W,), jnp.int32),
            pltpu.VMEM((K, DC), jnp.float32),
            pltpu.VMEM((16,), jnp.int32),
            pltpu.VMEM((16,), jnp.int32),
            pltpu.SemaphoreType.DMA,
        ],
    )
    return f(table2, sidx, idx, clstab)


def kernel(indices, input_tensor, classes_arr):
    idx = indices.astype(jnp.int32)
    sidx = (idx[:, None] * C + jnp.arange(C, dtype=jnp.int32)[None, :]).reshape(-1)
    table2 = input_tensor.reshape(N_CLS * C, DC)
    clstab = jnp.pad(classes_arr.astype(jnp.int32), (0, 128 - N_CLS))
    out2, cls = _run(table2, sidx, idx, clstab)
    return out2.reshape(B, 3, 224, 224), cls


# speedup vs baseline: 1.8218x; 1.8218x over previous
import jax
import jax.numpy as jnp
from jax import lax
from jax.experimental import pallas as pl
from jax.experimental.pallas import tpu as pltpu
from jax.experimental.pallas import tpu_sc as plsc

N_CLS = 100
IMG_ELEMS = 3 * 224 * 224
B = 256
C = 24
DC = IMG_ELEMS // C
NW = 32
ROWS_PER_W = (B * C) // NW
K = 8
GROUPS = ROWS_PER_W // K
VECS = DC // 16
NBUF = 2


def _sc_body(table_hbm, sidx_hbm, idx_hbm, clstab_hbm, out_hbm, cls_hbm,
             sidx_v, buf, idx16_v, cls_v,
             sem_in0, sem_in1, sem_out0, sem_out1, sem_cls):
    c = lax.axis_index("c")
    s = lax.axis_index("s")
    wid = s * 2 + c
    sems_in = (sem_in0, sem_in1)
    sems_out = (sem_out0, sem_out1)

    @pl.when(wid < 16)
    def _():
        pltpu.sync_copy(idx_hbm.at[pl.ds(wid * 16, 16)], idx16_v)
        pltpu.async_copy(clstab_hbm.at[idx16_v], cls_v, sem_cls).wait()
        pltpu.sync_copy(cls_v, cls_hbm.at[pl.ds(wid * 16, 16)])

    base = wid * ROWS_PER_W
    pltpu.sync_copy(sidx_hbm.at[pl.ds(wid * GROUPS, GROUPS)], sidx_v)

    def compute(b):
        def row_body(r, _):
            @plsc.parallel_loop(0, VECS, unroll=8)
            def _vec(i):
                x = buf[b, r, pl.ds(i * 16, 16)]
                buf[b, r, pl.ds(i * 16, 16)] = (
                    jnp.minimum(jnp.maximum(x, -0.5), 0.5) + 0.5
                )
            return 0
        lax.fori_loop(0, K, row_body, 0)

    cps_in = [None, None]
    cps_out = [None, None]
    cps_in[0] = pltpu.async_copy(
        table_hbm.at[sidx_v.at[0]], buf.at[0], sems_in[0])
    for g in range(GROUPS):
        b = g & 1
        nb = b ^ 1
        if g + 1 < GROUPS:
            if g >= 1:
                cps_out[nb].wait()
            cps_in[nb] = pltpu.async_copy(
                table_hbm.at[sidx_v.at[g + 1]], buf.at[nb], sems_in[nb])
        cps_in[b].wait()
        compute(b)
        cps_out[b] = pltpu.async_copy(
            buf.at[b], out_hbm.at[pl.ds(base + g * K, K)], sems_out[b])
    cps_out[0].wait()
    cps_out[1].wait()


@jax.jit
def _run(table2, sidx, idx, clstab):
    mesh = plsc.VectorSubcoreMesh(
        core_axis_name="c", subcore_axis_name="s", num_cores=2, num_subcores=16
    )
    f = pl.kernel(
        _sc_body,
        out_type=(
            jax.ShapeDtypeStruct((B * C, DC), jnp.float32),
            jax.ShapeDtypeStruct((B,), jnp.int32),
        ),
        mesh=mesh,
        scratch_types=[
            pltpu.VMEM((GROUPS, K), jnp.int32),
            pltpu.VMEM((NBUF, K, DC), jnp.float32),
            pltpu.VMEM((16,), jnp.int32),
            pltpu.VMEM((16,), jnp.int32),
            pltpu.SemaphoreType.DMA,
            pltpu.SemaphoreType.DMA,
            pltpu.SemaphoreType.DMA,
            pltpu.SemaphoreType.DMA,
            pltpu.SemaphoreType.DMA,
        ],
    )
    return f(table2, sidx, idx, clstab)


def kernel(indices, input_tensor, classes_arr):
    idx = indices.astype(jnp.int32)
    sidx = (idx[:, None] * C
            + jnp.arange(C, dtype=jnp.int32)[None, :]).reshape(NW * GROUPS, K)
    table2 = input_tensor.reshape(N_CLS * C, DC)
    clstab = jnp.pad(classes_arr.astype(jnp.int32), (0, 128 - N_CLS))
    out2, cls = _run(table2, sidx, idx, clstab)
    return out2.reshape(B, 3, 224, 224), cls


# speedup vs baseline: 3.3770x; 1.8537x over previous
import jax
import jax.numpy as jnp
from jax import lax
from jax.experimental import pallas as pl
from jax.experimental.pallas import tpu as pltpu
from jax.experimental.pallas import tpu_sc as plsc

N_CLS = 100
B = 256
H = 224
W = 224
NPLANES = B * 3
NW = 32
GROUPS = NPLANES // NW
NVEC = W // 16
NBUF = 2


def _sc_body(table_hbm, sidx_hbm, idx_hbm, clstab_hbm, out_hbm, cls_hbm,
             sidx_v, buf, idx16_v, cls_v,
             sem_in0, sem_in1, sem_out0, sem_out1, sem_cls):
    c = lax.axis_index("c")
    s = lax.axis_index("s")
    wid = s * 2 + c
    sems_in = (sem_in0, sem_in1)
    sems_out = (sem_out0, sem_out1)

    @pl.when(wid < 16)
    def _():
        pltpu.sync_copy(idx_hbm.at[pl.ds(wid * 16, 16)], idx16_v)
        pltpu.async_copy(clstab_hbm.at[idx16_v], cls_v, sem_cls).wait()
        pltpu.sync_copy(cls_v, cls_hbm.at[pl.ds(wid * 16, 16)])

    base = wid * GROUPS
    pltpu.sync_copy(sidx_hbm.at[wid], sidx_v)
    lanes = jnp.arange(16, dtype=jnp.int32)

    def compute(b):
        @plsc.parallel_loop(0, H, unroll=2)
        def _row(r):
            for cc in range(NVEC):
                x = buf[b, r, pl.ds(cc * 16, 16)]
                buf[b, r, pl.ds(cc * 16, 16)] = (
                    jnp.minimum(jnp.maximum(x, -0.5), 0.5) + 0.5
                )

    def gather(g, b):
        v = sidx_v[pl.ds((g // 16) * 16, 16)]
        row = v[g % 16]
        return pltpu.async_copy(table_hbm.at[row], buf.at[b], sems_in[b])

    cps_in = [None, None]
    cps_out = [None, None]
    cps_in[0] = gather(0, 0)
    for g in range(GROUPS):
        b = g & 1
        nb = b ^ 1
        if g + 1 < GROUPS:
            if g >= 1:
                cps_out[nb].wait()
            cps_in[nb] = gather(g + 1, nb)
        cps_in[b].wait()
        compute(b)
        cps_out[b] = pltpu.async_copy(
            buf.at[b], out_hbm.at[base + g], sems_out[b])
    cps_out[0].wait()
    cps_out[1].wait()


@jax.jit
def _run(table3, sidx, idx, clstab):
    mesh = plsc.VectorSubcoreMesh(
        core_axis_name="c", subcore_axis_name="s", num_cores=2, num_subcores=16
    )
    f = pl.kernel(
        _sc_body,
        out_type=(
            jax.ShapeDtypeStruct((NPLANES, H, W), jnp.float32),
            jax.ShapeDtypeStruct((B,), jnp.int32),
        ),
        mesh=mesh,
        scratch_types=[
            pltpu.VMEM((32,), jnp.int32),
            pltpu.VMEM((NBUF, H, W), jnp.float32),
            pltpu.VMEM((16,), jnp.int32),
            pltpu.VMEM((16,), jnp.int32),
            pltpu.SemaphoreType.DMA,
            pltpu.SemaphoreType.DMA,
            pltpu.SemaphoreType.DMA,
            pltpu.SemaphoreType.DMA,
            pltpu.SemaphoreType.DMA,
        ],
    )
    return f(table3, sidx, idx, clstab)


def kernel(indices, input_tensor, classes_arr):
    idx = indices.astype(jnp.int32)
    sidx = (idx[:, None] * 3
            + jnp.arange(3, dtype=jnp.int32)[None, :]).reshape(NW, GROUPS)
    sidx = jnp.pad(sidx, ((0, 0), (0, 32 - GROUPS)))
    table3 = input_tensor.reshape(N_CLS * 3, H, W)
    clstab = jnp.pad(classes_arr.astype(jnp.int32), (0, 128 - N_CLS))
    out3, cls = _run(table3, sidx, idx, clstab)
    return out3.reshape(B, 3, H, W), cls
